# baseline (speedup 1.0000x reference)
import functools

import jax
import jax.numpy as jnp
from jax import lax
from jax.experimental import pallas as pl
from jax.experimental.pallas import tpu as pltpu, tpu_sc as plsc

B = 16384
F = 100
FPAD = 128
D = 128
N = 100000

NC = 2
NS = 16
L = 16
NW = NC * NS
ROWS_PER_W = B // NW
GROUPS = ROWS_PER_W // L
FCHUNKS = (16, 16, 16, 16, 16, 20)
GSP = 20
WSP = 104
DC = D // L

_PGRID = 8
_BBLK = B // _PGRID
_NBLK = 4000


def _pad_body(fv_ref, fvp_ref):
    fvp_ref[...] = jnp.pad(fv_ref[...], ((0, 0), (0, FPAD - F)))


_padx = pl.pallas_call(
    _pad_body,
    grid=(_PGRID,),
    in_specs=[pl.BlockSpec((_BBLK, F), lambda i: (i, 0))],
    out_specs=pl.BlockSpec((_BBLK, FPAD), lambda i: (i, 0)),
    out_shape=jax.ShapeDtypeStruct((B, FPAD), jnp.float32),
)


def _fm_body(fi_hbm, fv_hbm, bias_hbm, lw_hbm, vv_hbm, out_hbm,
             idx_v, x_v, x_nv, g_v0, g_v1, w_v, bias_v, s_v, q_v, out_v,
             sem0, sem1, sem_wn, sem_pf):
    wid = lax.axis_index("s") * NC + lax.axis_index("c")
    row_base = wid * ROWS_PER_W

    iota = lax.iota(jnp.int32, L)
    ivw = iota * WSP

    gbuf = (g_v0, g_v1)
    sems = (sem0, sem1)
    starts = []
    acc = 0
    for cs in FCHUNKS:
        starts.append(acc)
        acc += cs

    pltpu.sync_copy(bias_hbm, bias_v)
    bias_vec = bias_v[...]

    def fire(c):
        f0, cs = starts[c], FCHUNKS[c]
        g_v, sem = gbuf[c % 2], sems[c % 2]
        return [pltpu.async_copy(
                    vv_hbm.at[idx_v.at[l, pl.ds(f0, cs)]],
                    g_v.at[pl.ds(l * GSP, cs), :], sem)
                for l in range(L)]

    def fire_wn():
        return [pltpu.async_copy(
                    lw_hbm.at[idx_v.at[l, pl.ds(0, F)]],
                    w_v.at[pl.ds(l * WSP, F)], sem_wn)
                for l in range(L)]

    def rwait_c0():
        cs = FCHUNKS[0]
        for l in range(L):
            pltpu.make_async_copy(
                vv_hbm.at[pl.ds(0, cs), :],
                g_v0.at[pl.ds(l * GSP, cs), :], sem0).wait()

    def rwait_wn():
        for l in range(L):
            pltpu.make_async_copy(
                lw_hbm.at[pl.ds(0, F)], w_v.at[pl.ds(l * WSP, F)],
                sem_wn).wait()

    def group_body(g, carry):
        row0 = row_base + g * L
        lin = jnp.zeros((L,), jnp.float32)

        inflight = []
        for c, cs in enumerate(FCHUNKS):
            f0 = starts[c]
            g_v = gbuf[c % 2]
            if c + 1 < len(FCHUNKS):
                nxt = fire(c + 1)
            else:
                nxt = []
            if c == 0:
                rwait_c0()
            else:
                for cp in inflight:
                    cp.wait()
            inflight = nxt

            if c == len(FCHUNKS) - 1:
                nrow0 = row_base + ((g + 1) & (GROUPS - 1)) * L
                pf = [pltpu.async_copy(fi_hbm.at[pl.ds(nrow0, L)], idx_v,
                                       sem_pf),
                      pltpu.async_copy(fv_hbm.at[pl.ds(nrow0, L)], x_nv,
                                       sem_pf)]

            def row_body(r, carry2, c=c, cs=cs, f0=f0):
                if c == 0:
                    s = [jnp.zeros((L,), jnp.float32) for _ in range(DC)]
                    q = jnp.zeros((L,), jnp.float32)
                else:
                    s = [s_v[r, pl.ds(cc * L, L)] for cc in range(DC)]
                    q = q_v[r]
                xblk = {k: x_v[r, pl.ds(k * L, L)]
                        for k in range(f0 // L, (f0 + cs - 1) // L + 1)}
                for j in range(cs):
                    xb = lax.broadcast(xblk[(f0 + j) // L][(f0 + j) % L], (L,))
                    gr = r * GSP + j
                    for cc in range(DC):
                        p = xb * g_v[gr, pl.ds(cc * L, L)]
                        s[cc] = s[cc] + p
                        q = q + p * p
                for cc in range(DC):
                    s_v[r, pl.ds(cc * L, L)] = s[cc]
                q_v[r] = q
                return carry2

            lax.fori_loop(0, L, row_body, None)

            if c == 0:
                rwait_wn()
                for j in range(F):
                    xsj = plsc.load_gather(
                        x_v, [iota, jnp.full((L,), j, jnp.int32)])
                    wvec = plsc.load_gather(w_v, [ivw + j])
                    lin = lin + xsj * wvec

        def epi_body(r, acc):
            sr = [s_v[r, pl.ds(cc * L, L)] for cc in range(DC)]
            ssqv = sr[0] * sr[0]
            for cc in range(1, DC):
                ssqv = ssqv + sr[cc] * sr[cc]
            inter = 0.5 * (jnp.sum(ssqv) - jnp.sum(q_v[r]))
            return jnp.where(iota == r, lax.broadcast(inter, (L,)), acc)

        inter_vec = lax.fori_loop(0, L, epi_body, jnp.zeros((L,), jnp.float32))
        z = bias_vec + lin + inter_vec
        out_v[pl.ds(g * L, L)] = 1.0 / (1.0 + jnp.exp(-z))

        for cp in pf:
            cp.wait()

        def xcopy_body(r, carry3):
            for k in range(7):
                x_v[r, pl.ds(k * L, L)] = x_nv[r, pl.ds(k * L, L)]
            return carry3

        lax.fori_loop(0, L, xcopy_body, None)
        fire(0)
        fire_wn()
        return carry

    pltpu.sync_copy(fi_hbm.at[pl.ds(row_base, L)], idx_v)
    pltpu.sync_copy(fv_hbm.at[pl.ds(row_base, L)], x_v)
    fire(0)
    fire_wn()

    lax.fori_loop(0, GROUPS, group_body, None)

    rwait_c0()
    rwait_wn()

    pltpu.sync_copy(out_v, out_hbm.at[pl.ds(row_base, ROWS_PER_W)])


_fm = functools.partial(
    pl.kernel,
    out_type=jax.ShapeDtypeStruct((B,), jnp.float32),
    mesh=plsc.VectorSubcoreMesh(core_axis_name="c", subcore_axis_name="s"),
    compiler_params=pltpu.CompilerParams(needs_layout_passes=False),
    scratch_types=[
        pltpu.VMEM((L, F), jnp.int32),
        pltpu.VMEM((L, FPAD), jnp.float32),
        pltpu.VMEM((L, FPAD), jnp.float32),
        pltpu.VMEM((L * GSP, D), jnp.float32),
        pltpu.VMEM((L * GSP, D), jnp.float32),
        pltpu.VMEM((L * WSP,), jnp.float32),
        pltpu.VMEM((L,), jnp.float32),
        pltpu.VMEM((L, D), jnp.float32),
        pltpu.VMEM((L, L), jnp.float32),
        pltpu.VMEM((ROWS_PER_W,), jnp.float32),
        pltpu.SemaphoreType.DMA,
        pltpu.SemaphoreType.DMA,
        pltpu.SemaphoreType.DMA,
        pltpu.SemaphoreType.DMA,
    ],
)(_fm_body)


def kernel(feature_indices, feature_values, bias, linear_w, factor_v):
    fi = feature_indices.astype(jnp.int32)
    fv = jnp.pad(feature_values.astype(jnp.float32),
                 ((0, 0), (0, FPAD - F)))
    bias16 = jnp.broadcast_to(bias.reshape(()), (L,))
    return _fm(fi, fv, bias16, linear_w.reshape(N), factor_v)


# speedup vs baseline: 1.2921x; 1.2921x over previous
import functools

import jax
import jax.numpy as jnp
from jax import lax
from jax.experimental import pallas as pl
from jax.experimental.pallas import tpu as pltpu, tpu_sc as plsc

B = 16384
F = 100
FPAD = 128
D = 128
N = 100000

NC = 2
NS = 16
L = 16
NW = NC * NS
ROWS_PER_W = B // NW
GROUPS = ROWS_PER_W // L
FCHUNKS = (16, 16, 16, 16, 16, 20)
GSP = 20
WSP = 104
DC = D // L

_PGRID = 8
_BBLK = B // _PGRID
_NBLK = 20000


def _pad_body(fv_ref, fvp_ref):
    fvp_ref[...] = jnp.pad(fv_ref[...], ((0, 0), (0, FPAD - F)))


_padx = pl.pallas_call(
    _pad_body,
    grid=(_PGRID,),
    in_specs=[pl.BlockSpec((_BBLK, F), lambda i: (i, 0))],
    out_specs=pl.BlockSpec((_BBLK, FPAD), lambda i: (i, 0)),
    out_shape=jax.ShapeDtypeStruct((B, FPAD), jnp.float32),
)


def _norm_body(v_ref, n_ref):
    v = v_ref[...]
    n_ref[...] = jnp.sum(v * v, axis=2)


_rownorm = pl.pallas_call(
    _norm_body,
    grid=(N // _NBLK,),
    in_specs=[pl.BlockSpec((_NBLK // 4, 4, D), lambda i: (i, 0, 0))],
    out_specs=pl.BlockSpec((_NBLK // 4, 4), lambda i: (i, 0)),
    out_shape=jax.ShapeDtypeStruct((N // 4, 4), jnp.float32),
)


def _fm_body(fi_hbm, fv_hbm, bias_hbm, lw_hbm, n_hbm, vv_hbm, out_hbm,
             idx_v, x_v, x_nv, g_v0, g_v1, w_v, n_v, bias_v, s_v, out_v,
             sem0, sem1, sem_wn, sem_pf):
    wid = lax.axis_index("s") * NC + lax.axis_index("c")
    row_base = wid * ROWS_PER_W

    iota = lax.iota(jnp.int32, L)
    ivw = iota * WSP

    gbuf = (g_v0, g_v1)
    sems = (sem0, sem1)
    starts = []
    acc = 0
    for cs in FCHUNKS:
        starts.append(acc)
        acc += cs

    pltpu.sync_copy(bias_hbm, bias_v)
    bias_vec = bias_v[...]

    def fire(c):
        f0, cs = starts[c], FCHUNKS[c]
        g_v, sem = gbuf[c % 2], sems[c % 2]
        return [pltpu.async_copy(
                    vv_hbm.at[idx_v.at[l, pl.ds(f0, cs)]],
                    g_v.at[pl.ds(l * GSP, cs), :], sem)
                for l in range(L)]

    def fire_wn():
        return [pltpu.async_copy(
                    (lw_hbm if k == 0 else n_hbm).at[idx_v.at[l, pl.ds(0, F)]],
                    (w_v if k == 0 else n_v).at[pl.ds(l * WSP, F)], sem_wn)
                for l in range(L) for k in range(2)]

    def rwait_c0():
        cs = FCHUNKS[0]
        for l in range(L):
            pltpu.make_async_copy(
                vv_hbm.at[pl.ds(0, cs), :],
                g_v0.at[pl.ds(l * GSP, cs), :], sem0).wait()

    def rwait_wn():
        for l in range(L):
            pltpu.make_async_copy(
                lw_hbm.at[pl.ds(0, F)], w_v.at[pl.ds(l * WSP, F)],
                sem_wn).wait()
            pltpu.make_async_copy(
                n_hbm.at[pl.ds(0, F)], n_v.at[pl.ds(l * WSP, F)],
                sem_wn).wait()

    def group_body(g, carry):
        row0 = row_base + g * L
        lin = jnp.zeros((L,), jnp.float32)
        qn = jnp.zeros((L,), jnp.float32)

        inflight = []
        for c, cs in enumerate(FCHUNKS):
            f0 = starts[c]
            g_v = gbuf[c % 2]
            if c + 1 < len(FCHUNKS):
                nxt = fire(c + 1)
            else:
                nxt = []
            if c == 0:
                rwait_c0()
            else:
                for cp in inflight:
                    cp.wait()
            inflight = nxt

            if c == len(FCHUNKS) - 1:
                nrow0 = row_base + ((g + 1) & (GROUPS - 1)) * L
                pf = [pltpu.async_copy(fi_hbm.at[pl.ds(nrow0, L)], idx_v,
                                       sem_pf),
                      pltpu.async_copy(fv_hbm.at[pl.ds(nrow0, L)], x_nv,
                                       sem_pf)]

            def row_body(r, carry2, c=c, cs=cs, f0=f0):
                if c == 0:
                    s = [jnp.zeros((L,), jnp.float32) for _ in range(DC)]
                else:
                    s = [s_v[r, pl.ds(cc * L, L)] for cc in range(DC)]
                xblk = {k: x_v[r, pl.ds(k * L, L)]
                        for k in range(f0 // L, (f0 + cs - 1) // L + 1)}
                for j in range(cs):
                    xb = lax.broadcast(xblk[(f0 + j) // L][(f0 + j) % L], (L,))
                    gr = r * GSP + j
                    for cc in range(DC):
                        s[cc] = s[cc] + xb * g_v[gr, pl.ds(cc * L, L)]
                for cc in range(DC):
                    s_v[r, pl.ds(cc * L, L)] = s[cc]
                return carry2

            lax.fori_loop(0, L, row_body, None)

            if c == 0:
                rwait_wn()
                for j in range(F):
                    xsj = plsc.load_gather(
                        x_v, [iota, jnp.full((L,), j, jnp.int32)])
                    wvec = plsc.load_gather(w_v, [ivw + j])
                    nvec = plsc.load_gather(n_v, [ivw + j])
                    lin = lin + xsj * wvec
                    qn = qn + (xsj * xsj) * nvec

        def epi_body(r, acc):
            sr = [s_v[r, pl.ds(cc * L, L)] for cc in range(DC)]
            ssqv = sr[0] * sr[0]
            for cc in range(1, DC):
                ssqv = ssqv + sr[cc] * sr[cc]
            inter = 0.5 * jnp.sum(ssqv)
            return jnp.where(iota == r, lax.broadcast(inter, (L,)), acc)

        inter_vec = lax.fori_loop(0, L, epi_body, jnp.zeros((L,), jnp.float32))
        z = bias_vec + lin + inter_vec - 0.5 * qn
        out_v[pl.ds(g * L, L)] = 1.0 / (1.0 + jnp.exp(-z))

        for cp in pf:
            cp.wait()

        def xcopy_body(r, carry3):
            for k in range(7):
                x_v[r, pl.ds(k * L, L)] = x_nv[r, pl.ds(k * L, L)]
            return carry3

        lax.fori_loop(0, L, xcopy_body, None)
        fire(0)
        fire_wn()
        return carry

    pltpu.sync_copy(fi_hbm.at[pl.ds(row_base, L)], idx_v)
    pltpu.sync_copy(fv_hbm.at[pl.ds(row_base, L)], x_v)
    fire(0)
    fire_wn()

    lax.fori_loop(0, GROUPS, group_body, None)

    rwait_c0()
    rwait_wn()

    pltpu.sync_copy(out_v, out_hbm.at[pl.ds(row_base, ROWS_PER_W)])


_fm = functools.partial(
    pl.kernel,
    out_type=jax.ShapeDtypeStruct((B,), jnp.float32),
    mesh=plsc.VectorSubcoreMesh(core_axis_name="c", subcore_axis_name="s"),
    compiler_params=pltpu.CompilerParams(needs_layout_passes=False),
    scratch_types=[
        pltpu.VMEM((L, F), jnp.int32),
        pltpu.VMEM((L, FPAD), jnp.float32),
        pltpu.VMEM((L, FPAD), jnp.float32),
        pltpu.VMEM((L * GSP, D), jnp.float32),
        pltpu.VMEM((L * GSP, D), jnp.float32),
        pltpu.VMEM((L * WSP,), jnp.float32),
        pltpu.VMEM((L * WSP,), jnp.float32),
        pltpu.VMEM((L,), jnp.float32),
        pltpu.VMEM((L, D), jnp.float32),
        pltpu.VMEM((ROWS_PER_W,), jnp.float32),
        pltpu.SemaphoreType.DMA,
        pltpu.SemaphoreType.DMA,
        pltpu.SemaphoreType.DMA,
        pltpu.SemaphoreType.DMA,
    ],
)(_fm_body)


def kernel(feature_indices, feature_values, bias, linear_w, factor_v):
    fi = feature_indices.astype(jnp.int32)
    fv = jnp.pad(feature_values.astype(jnp.float32),
                 ((0, 0), (0, FPAD - F)))
    bias16 = jnp.broadcast_to(bias.reshape(()), (L,))
    norms = _rownorm(factor_v.reshape(N // 4, 4, D)).reshape(N)
    return _fm(fi, fv, bias16, linear_w.reshape(N), norms, factor_v)


# speedup vs baseline: 1.2934x; 1.0010x over previous
import functools

import jax
import jax.numpy as jnp
from jax import lax
from jax.experimental import pallas as pl
from jax.experimental.pallas import tpu as pltpu, tpu_sc as plsc

B = 16384
F = 100
FPAD = 128
D = 128
N = 100000

NC = 2
NS = 16
L = 16
NW = NC * NS
ROWS_PER_W = B // NW
GROUPS = ROWS_PER_W // L
FCHUNKS = (16, 16, 16, 16, 16, 20)
GSP = 20
WSP = 104
DC = D // L

_NBLK = 20000


def _norm_body(v_ref, n_ref):
    v = v_ref[...]
    n_ref[...] = jnp.sum(v * v, axis=2)


_rownorm = pl.pallas_call(
    _norm_body,
    grid=(N // _NBLK,),
    in_specs=[pl.BlockSpec((_NBLK // 4, 4, D), lambda i: (i, 0, 0))],
    out_specs=pl.BlockSpec((_NBLK // 4, 4), lambda i: (i, 0)),
    out_shape=jax.ShapeDtypeStruct((N // 4, 4), jnp.float32),
)


def _fm_body(fi_hbm, fv_hbm, bias_hbm, lw_hbm, n_hbm, vv_hbm, out_hbm,
             idx_v, x_v, x_nv, g_v0, g_v1, w_v, n_v, bias_v, s_v, out_v,
             sem0, sem1, sem_wn, sem_pf):
    wid = lax.axis_index("s") * NC + lax.axis_index("c")
    row_base = wid * ROWS_PER_W

    iota = lax.iota(jnp.int32, L)
    ivw = iota * WSP

    gbuf = (g_v0, g_v1)
    sems = (sem0, sem1)
    starts = []
    acc = 0
    for cs in FCHUNKS:
        starts.append(acc)
        acc += cs

    pltpu.sync_copy(bias_hbm, bias_v)
    bias_vec = bias_v[...]

    def fire(c):
        f0, cs = starts[c], FCHUNKS[c]
        g_v, sem = gbuf[c % 2], sems[c % 2]
        return [pltpu.async_copy(
                    vv_hbm.at[idx_v.at[l, pl.ds(f0, cs)]],
                    g_v.at[pl.ds(l * GSP, cs), :], sem)
                for l in range(L)]

    def fire_wn():
        return [pltpu.async_copy(
                    (lw_hbm if k == 0 else n_hbm).at[idx_v.at[l, pl.ds(0, F)]],
                    (w_v if k == 0 else n_v).at[pl.ds(l * WSP, F)], sem_wn)
                for l in range(L) for k in range(2)]

    def rwait_c0():
        cs = FCHUNKS[0]
        for l in range(L):
            pltpu.make_async_copy(
                vv_hbm.at[pl.ds(0, cs), :],
                g_v0.at[pl.ds(l * GSP, cs), :], sem0).wait()

    def rwait_wn():
        for l in range(L):
            pltpu.make_async_copy(
                lw_hbm.at[pl.ds(0, F)], w_v.at[pl.ds(l * WSP, F)],
                sem_wn).wait()
            pltpu.make_async_copy(
                n_hbm.at[pl.ds(0, F)], n_v.at[pl.ds(l * WSP, F)],
                sem_wn).wait()

    def group_body(g, carry):
        row0 = row_base + g * L
        lin = jnp.zeros((L,), jnp.float32)
        qn = jnp.zeros((L,), jnp.float32)

        inflight = []
        for c, cs in enumerate(FCHUNKS):
            f0 = starts[c]
            g_v = gbuf[c % 2]
            if c + 1 < len(FCHUNKS):
                nxt = fire(c + 1)
            else:
                nxt = []
            if c == 0:
                rwait_c0()
            else:
                for cp in inflight:
                    cp.wait()
            inflight = nxt

            if c == len(FCHUNKS) - 1:
                nrow0 = row_base + ((g + 1) & (GROUPS - 1)) * L
                pf = [pltpu.async_copy(fi_hbm.at[pl.ds(nrow0, L)], idx_v,
                                       sem_pf),
                      pltpu.async_copy(fv_hbm.at[pl.ds(nrow0, L)], x_nv,
                                       sem_pf)]

            def row_body(r, carry2, c=c, cs=cs, f0=f0):
                if c == 0:
                    s = [jnp.zeros((L,), jnp.float32) for _ in range(DC)]
                else:
                    s = [s_v[r, pl.ds(cc * L, L)] for cc in range(DC)]
                xblk = {k: x_v[r, pl.ds(k * L, L)]
                        for k in range(f0 // L, (f0 + cs - 1) // L + 1)}
                for j in range(cs):
                    xb = lax.broadcast(xblk[(f0 + j) // L][(f0 + j) % L], (L,))
                    gr = r * GSP + j
                    for cc in range(DC):
                        s[cc] = s[cc] + xb * g_v[gr, pl.ds(cc * L, L)]
                for cc in range(DC):
                    s_v[r, pl.ds(cc * L, L)] = s[cc]
                return carry2

            lax.fori_loop(0, L, row_body, None)

            if c == 0:
                rwait_wn()
                for j in range(F):
                    xsj = plsc.load_gather(
                        x_v, [iota, jnp.full((L,), j, jnp.int32)])
                    wvec = plsc.load_gather(w_v, [ivw + j])
                    nvec = plsc.load_gather(n_v, [ivw + j])
                    lin = lin + xsj * wvec
                    qn = qn + (xsj * xsj) * nvec

        def epi_body(r, acc):
            sr = [s_v[r, pl.ds(cc * L, L)] for cc in range(DC)]
            ssqv = sr[0] * sr[0]
            for cc in range(1, DC):
                ssqv = ssqv + sr[cc] * sr[cc]
            inter = 0.5 * jnp.sum(ssqv)
            return jnp.where(iota == r, lax.broadcast(inter, (L,)), acc)

        inter_vec = lax.fori_loop(0, L, epi_body, jnp.zeros((L,), jnp.float32))
        z = bias_vec + lin + inter_vec - 0.5 * qn
        out_v[pl.ds(g * L, L)] = 1.0 / (1.0 + jnp.exp(-z))

        for cp in pf:
            cp.wait()

        def xcopy_body(r, carry3):
            for k in range(7):
                x_v[r, pl.ds(k * L, L)] = x_nv[r, pl.ds(k * L, L)]
            return carry3

        lax.fori_loop(0, L, xcopy_body, None)
        fire(0)
        fire_wn()
        return carry

    pltpu.sync_copy(fi_hbm.at[pl.ds(row_base, L)], idx_v)
    pltpu.sync_copy(fv_hbm.at[pl.ds(row_base, L)], x_v)
    fire(0)
    fire_wn()

    lax.fori_loop(0, GROUPS, group_body, None)

    rwait_c0()
    rwait_wn()

    pltpu.sync_copy(out_v, out_hbm.at[pl.ds(row_base, ROWS_PER_W)])


_fm = functools.partial(
    pl.kernel,
    out_type=jax.ShapeDtypeStruct((B,), jnp.float32),
    mesh=plsc.VectorSubcoreMesh(core_axis_name="c", subcore_axis_name="s"),
    compiler_params=pltpu.CompilerParams(needs_layout_passes=False),
    scratch_types=[
        pltpu.VMEM((L, F), jnp.int32),
        pltpu.VMEM((L, FPAD), jnp.float32),
        pltpu.VMEM((L, FPAD), jnp.float32),
        pltpu.VMEM((L * GSP, D), jnp.float32),
        pltpu.VMEM((L * GSP, D), jnp.float32),
        pltpu.VMEM((L * WSP,), jnp.float32),
        pltpu.VMEM((L * WSP,), jnp.float32),
        pltpu.VMEM((L,), jnp.float32),
        pltpu.VMEM((L, D), jnp.float32),
        pltpu.VMEM((ROWS_PER_W,), jnp.float32),
        pltpu.SemaphoreType.DMA,
        pltpu.SemaphoreType.DMA,
        pltpu.SemaphoreType.DMA,
        pltpu.SemaphoreType.DMA,
    ],
)(_fm_body)


def kernel(feature_indices, feature_values, bias, linear_w, factor_v):
    fi = feature_indices.astype(jnp.int32)
    fv = jnp.pad(feature_values.astype(jnp.float32),
                 ((0, 0), (0, FPAD - F)))
    bias16 = jnp.broadcast_to(bias.reshape(()), (L,))
    norms = _rownorm(factor_v.reshape(N // 4, 4, D)).reshape(N)
    return _fm(fi, fv, bias16, linear_w.reshape(N), norms, factor_v)
